# baseline (speedup 1.0000x reference)
import functools

import jax
import jax.numpy as jnp
from jax import lax
from jax.experimental import pallas as pl
from jax.experimental.pallas import tpu as pltpu
from jax.experimental.pallas import tpu_sc as plsc

_NC = 2
_NS = 16
_NW = _NC * _NS
_CHUNK = 128
_SENT = -1

_SC_PARAMS = pltpu.CompilerParams(
    use_tc_tiling_on_sc=True, needs_layout_passes=False)


def _mesh():
    return plsc.VectorSubcoreMesh(
        core_axis_name="c", subcore_axis_name="s",
        num_cores=_NC, num_subcores=_NS)


def _make_fused(m2, d2, b, rows_per_w, copy_chunk):
    n_copy = rows_per_w // copy_chunk
    tail = m2 - rows_per_w * _NW
    n_chunks = b // _CHUNK
    blocks = n_chunks // _NS
    half = _NS * rows_per_w

    @functools.partial(
        pl.kernel,
        mesh=_mesh(),
        compiler_params=_SC_PARAMS,
        out_type=jax.ShapeDtypeStruct((m2, d2), jnp.float32),
        scratch_types=[
            pltpu.VMEM((copy_chunk, d2), jnp.float32),
            pltpu.VMEM((copy_chunk, d2), jnp.float32),
            pltpu.VMEM((blocks, _CHUNK), jnp.int32),
            pltpu.VMEM((_CHUNK,), jnp.int32),
            pltpu.VMEM((_CHUNK, d2), jnp.float32),
            pltpu.SemaphoreType.DMA,
            pltpu.SemaphoreType.DMA,
            pltpu.SemaphoreType.DMA,
            pltpu.SemaphoreType.DMA,
            pltpu.SemaphoreType.DMA,
        ],
    )
    def fused(orig_hbm, pidx_hbm, prow_hbm, out_ref,
              buf0, buf1, didx, midx, rows,
              rs0, rs1, ws0, ws1, ssem):
        core = lax.axis_index("c")
        sub = lax.axis_index("s")
        wid = core * _NS + sub
        base = wid * rows_per_w
        bufs = (buf0, buf1)
        rsems = (rs0, rs1)
        wsems = (ws0, ws1)

        def rd(c):
            return pltpu.make_async_copy(
                orig_hbm.at[pl.ds(base + c * copy_chunk, copy_chunk)],
                bufs[c % 2], rsems[c % 2])

        def wr(c):
            return pltpu.make_async_copy(
                bufs[c % 2],
                out_ref.at[pl.ds(base + c * copy_chunk, copy_chunk)],
                wsems[c % 2])

        rd(0).start()
        for c in range(n_copy):
            if c + 1 < n_copy:
                if c >= 1:
                    wr(c - 1).wait()
                rd(c + 1).start()
            rd(c).wait()
            wr(c).start()
        if n_copy >= 2:
            wr(n_copy - 2).wait()
        wr(n_copy - 1).wait()

        if tail:
            @pl.when(wid == _NW - 1)
            def _():
                t = pltpu.make_async_copy(
                    orig_hbm.at[pl.ds(rows_per_w * _NW, tail)],
                    bufs[0].at[pl.ds(0, tail)], rsems[0])
                t.start()
                t.wait()
                t2 = pltpu.make_async_copy(
                    bufs[0].at[pl.ds(0, tail)],
                    out_ref.at[pl.ds(rows_per_w * _NW, tail)], wsems[0])
                t2.start()
                t2.wait()

        plsc.subcore_barrier()

        lo = core * half
        hi = half + core * (m2 - half)

        pltpu.sync_copy(pidx_hbm.at[sub], didx)
        for q in range(blocks):
            j = sub * blocks + q
            for g in range(_CHUNK // 16):
                dv = didx[q, pl.ds(g * 16, 16)]
                keep = (dv >= lo) & (dv < hi)
                midx[pl.ds(g * 16, 16)] = jnp.where(keep, dv, _SENT)
            pltpu.sync_copy(prow_hbm.at[pl.ds(j * _CHUNK, _CHUNK)], rows)
            pltpu.async_copy(
                rows, out_ref.at[plsc.Indices(midx, ignored_value=_SENT)],
                ssem).wait()

    return fused


def kernel(mat_orig, indices, mat_new):
    m, d = mat_orig.shape
    b = indices.shape[0]
    m2, d2 = m // 2, d * 2
    rows_per_w = (m2 // _NW) // 8 * 8
    copy_chunk = 248
    assert rows_per_w % copy_chunk == 0

    idx = indices.astype(jnp.int32)
    pos = jnp.arange(b, dtype=jnp.int32)
    wpos = jnp.full((m,), -1, jnp.int32).at[idx].max(pos)

    pair = idx >> 1
    w0 = wpos[pair * 2]
    w1 = wpos[pair * 2 + 1]
    left = jnp.where((w0 >= 0)[:, None],
                     mat_new[jnp.maximum(w0, 0)], mat_orig[pair * 2])
    right = jnp.where((w1 >= 0)[:, None],
                      mat_new[jnp.maximum(w1, 0)], mat_orig[pair * 2 + 1])
    prow = jnp.concatenate([left, right], axis=1)

    pidx3 = pair.reshape(_NS, b // (_NS * _CHUNK), _CHUNK)

    fused = _make_fused(m2, d2, b, rows_per_w, copy_chunk)
    out128 = fused(mat_orig.reshape(m2, d2), pidx3, prow)
    return out128.reshape(m, d)


# speedup vs baseline: 1.5869x; 1.5869x over previous
import functools

import jax
import jax.numpy as jnp
from jax import lax
from jax.experimental import pallas as pl
from jax.experimental.pallas import tpu as pltpu
from jax.experimental.pallas import tpu_sc as plsc

_NC = 2
_NS = 16
_NW = _NC * _NS

_SC_PARAMS = pltpu.CompilerParams(
    use_tc_tiling_on_sc=True, needs_layout_passes=False)


def _mesh():
    return plsc.VectorSubcoreMesh(
        core_axis_name="c", subcore_axis_name="s",
        num_cores=_NC, num_subcores=_NS)


def _make_fused(m, d, b, rows_per_w, copy_chunk):
    n_copy = rows_per_w // copy_chunk
    tail = m - rows_per_w * _NW
    per_tile = b // _NS
    half = _NS * rows_per_w

    @functools.partial(
        pl.kernel,
        mesh=_mesh(),
        compiler_params=_SC_PARAMS,
        out_type=jax.ShapeDtypeStruct((m, d), jnp.float32),
        scratch_types=[
            pltpu.VMEM((copy_chunk, d), jnp.float32),
            pltpu.VMEM((copy_chunk, d), jnp.float32),
            pltpu.VMEM((128,), jnp.int32),
            pltpu.VMEM((128,), jnp.int32),
            pltpu.SemaphoreType.DMA,
            pltpu.SemaphoreType.DMA,
            pltpu.SemaphoreType.DMA,
            pltpu.SemaphoreType.DMA,
            pltpu.SemaphoreType.DMA,
        ],
    )
    def fused(orig_hbm, idx_hbm, wv_hbm, new_hbm, out_ref,
              buf0, buf1, didx, wpv,
              rs0, rs1, ws0, ws1, ssem):
        core = lax.axis_index("c")
        sub = lax.axis_index("s")
        wid = core * _NS + sub
        base = wid * rows_per_w
        bufs = (buf0, buf1)
        rsems = (rs0, rs1)
        wsems = (ws0, ws1)

        def rd(c):
            return pltpu.make_async_copy(
                orig_hbm.at[pl.ds(base + c * copy_chunk, copy_chunk)],
                bufs[c % 2], rsems[c % 2])

        def wr(c):
            return pltpu.make_async_copy(
                bufs[c % 2],
                out_ref.at[pl.ds(base + c * copy_chunk, copy_chunk)],
                wsems[c % 2])

        rd(0).start()
        for c in range(n_copy):
            if c + 1 < n_copy:
                if c >= 1:
                    wr(c - 1).wait()
                rd(c + 1).start()
            rd(c).wait()
            wr(c).start()
        if n_copy >= 2:
            wr(n_copy - 2).wait()
        wr(n_copy - 1).wait()

        if tail:
            @pl.when(wid == _NW - 1)
            def _():
                t = pltpu.make_async_copy(
                    orig_hbm.at[pl.ds(rows_per_w * _NW, tail)],
                    bufs[0].at[pl.ds(0, tail)], rsems[0])
                t.start()
                t.wait()
                t2 = pltpu.make_async_copy(
                    bufs[0].at[pl.ds(0, tail)],
                    out_ref.at[pl.ds(rows_per_w * _NW, tail)], wsems[0])
                t2.start()
                t2.wait()

        plsc.subcore_barrier()

        lo = core * half
        hi = half + core * (m - half)

        ebase = sub * per_tile
        lanes = lax.iota(jnp.int32, 16)

        @pl.loop(0, per_tile // 128, init_carry=jnp.int32(0))
        def n_fired(q, carry):
            pltpu.sync_copy(idx_hbm.at[sub].at[q], didx)
            pltpu.sync_copy(wv_hbm.at[sub].at[q], wpv)
            for g in range(8):
                dv = didx[pl.ds(g * 16, 16)]
                wv = wpv[pl.ds(g * 16, 16)]
                mypos = ebase + q * 128 + g * 16 + lanes
                keep = (wv == mypos) & (dv >= lo) & (dv < hi)
                carry = carry + jnp.sum(jnp.where(keep, 1, 0))
                for l in range(16):
                    sel = lanes == l
                    keep_s = jnp.sum(jnp.where(keep & sel, 1, 0))
                    dst_s = jnp.sum(jnp.where(sel, dv, 0))
                    src_s = ebase + q * 128 + g * 16 + l

                    @pl.when(keep_s > 0)
                    def _():
                        pltpu.async_copy(
                            new_hbm.at[pl.ds(src_s, 1)],
                            out_ref.at[pl.ds(dst_s, 1)], ssem)
            return carry

        @pl.loop(0, n_fired)
        def _(_i):
            pltpu.make_async_copy(
                new_hbm.at[pl.ds(0, 1)], out_ref.at[pl.ds(0, 1)], ssem).wait()

    return fused


def kernel(mat_orig, indices, mat_new):
    m, d = mat_orig.shape
    b = indices.shape[0]
    rows_per_w = (m // _NW) // 8 * 8
    copy_chunk = 248
    assert rows_per_w % copy_chunk == 0

    idx = indices.astype(jnp.int32)
    pos = jnp.arange(b, dtype=jnp.int32)
    wpos = jnp.full((m,), -1, jnp.int32).at[idx].max(pos)
    wvals = wpos[idx]

    per_tile = b // _NS
    idx3 = idx.reshape(_NS, per_tile // 128, 128)
    wv3 = wvals.reshape(_NS, per_tile // 128, 128)

    fused = _make_fused(m, d, b, rows_per_w, copy_chunk)
    return fused(mat_orig, idx3, wv3, mat_new)


# speedup vs baseline: 1.5894x; 1.0016x over previous
import functools

import jax
import jax.numpy as jnp
from jax import lax
from jax.experimental import pallas as pl
from jax.experimental.pallas import tpu as pltpu
from jax.experimental.pallas import tpu_sc as plsc

_NC = 2
_NS = 16
_NW = _NC * _NS

_SC_PARAMS = pltpu.CompilerParams(
    use_tc_tiling_on_sc=True, needs_layout_passes=False)


def _mesh():
    return plsc.VectorSubcoreMesh(
        core_axis_name="c", subcore_axis_name="s",
        num_cores=_NC, num_subcores=_NS)


def _make_fused(m, d, b, rows_per_w, copy_chunk):
    n_copy = rows_per_w // copy_chunk
    tail = m - rows_per_w * _NW
    per_tile = b // _NS
    half = _NS * rows_per_w

    @functools.partial(
        pl.kernel,
        mesh=_mesh(),
        compiler_params=_SC_PARAMS,
        out_type=jax.ShapeDtypeStruct((m, d), jnp.float32),
        scratch_types=[
            pltpu.VMEM((copy_chunk, d), jnp.float32),
            pltpu.VMEM((copy_chunk, d), jnp.float32),
            pltpu.VMEM((128,), jnp.int32),
            pltpu.VMEM((128,), jnp.int32),
            pltpu.SemaphoreType.DMA,
            pltpu.SemaphoreType.DMA,
            pltpu.SemaphoreType.DMA,
            pltpu.SemaphoreType.DMA,
            pltpu.SemaphoreType.DMA,
        ],
    )
    def fused(orig_hbm, idx_hbm, wv_hbm, new_hbm, out_ref,
              buf0, buf1, didx, wpv,
              rs0, rs1, ws0, ws1, ssem):
        core = lax.axis_index("c")
        sub = lax.axis_index("s")
        wid = core * _NS + sub
        base = wid * rows_per_w
        bufs = (buf0, buf1)
        rsems = (rs0, rs1)
        wsems = (ws0, ws1)

        def rd(c):
            return pltpu.make_async_copy(
                orig_hbm.at[pl.ds(base + c * copy_chunk, copy_chunk)],
                bufs[c % 2], rsems[c % 2])

        def wr(c):
            return pltpu.make_async_copy(
                bufs[c % 2],
                out_ref.at[pl.ds(base + c * copy_chunk, copy_chunk)],
                wsems[c % 2])

        rd(0).start()
        for c in range(n_copy):
            if c + 1 < n_copy:
                if c >= 1:
                    wr(c - 1).wait()
                rd(c + 1).start()
            rd(c).wait()
            wr(c).start()
        if n_copy >= 2:
            wr(n_copy - 2).wait()
        wr(n_copy - 1).wait()

        if tail:
            @pl.when(wid == _NW - 1)
            def _():
                t = pltpu.make_async_copy(
                    orig_hbm.at[pl.ds(rows_per_w * _NW, tail)],
                    bufs[0].at[pl.ds(0, tail)], rsems[0])
                t.start()
                t.wait()
                t2 = pltpu.make_async_copy(
                    bufs[0].at[pl.ds(0, tail)],
                    out_ref.at[pl.ds(rows_per_w * _NW, tail)], wsems[0])
                t2.start()
                t2.wait()

        plsc.subcore_barrier()

        lo = core * half
        hi = half + core * (m - half)

        ebase = sub * per_tile
        lanes = lax.iota(jnp.int32, 16)

        @pl.loop(0, per_tile // 128, init_carry=jnp.int32(0))
        def n_fired(q, carry):
            pltpu.sync_copy(idx_hbm.at[sub].at[q], didx)
            pltpu.sync_copy(wv_hbm.at[sub].at[q], wpv)
            for g in range(8):
                dv = didx[pl.ds(g * 16, 16)]
                wv = wpv[pl.ds(g * 16, 16)]
                mypos = ebase + q * 128 + g * 16 + lanes
                keep = (wv == mypos) & (dv >= lo) & (dv < hi)
                carry = carry + jnp.sum(jnp.where(keep, 1, 0))
                packed = jnp.where(keep, dv, -1)
                for l in range(16):
                    sel = lanes == l
                    dst_s = jnp.sum(jnp.where(sel, packed, 0))
                    src_s = ebase + q * 128 + g * 16 + l

                    @pl.when(dst_s >= 0)
                    def _():
                        pltpu.async_copy(
                            new_hbm.at[pl.ds(src_s, 1)],
                            out_ref.at[pl.ds(dst_s, 1)], ssem)
            return carry

        @pl.loop(0, n_fired)
        def _(_i):
            pltpu.make_async_copy(
                new_hbm.at[pl.ds(0, 1)], out_ref.at[pl.ds(0, 1)], ssem).wait()

    return fused


def kernel(mat_orig, indices, mat_new):
    m, d = mat_orig.shape
    b = indices.shape[0]
    rows_per_w = (m // _NW) // 8 * 8
    copy_chunk = 248
    assert rows_per_w % copy_chunk == 0

    idx = indices.astype(jnp.int32)
    pos = jnp.arange(b, dtype=jnp.int32)
    wpos = jnp.full((m,), -1, jnp.int32).at[idx].max(pos)
    wvals = wpos[idx]

    per_tile = b // _NS
    idx3 = idx.reshape(_NS, per_tile // 128, 128)
    wv3 = wvals.reshape(_NS, per_tile // 128, 128)

    fused = _make_fused(m, d, b, rows_per_w, copy_chunk)
    return fused(mat_orig, idx3, wv3, mat_new)
